# baseline (speedup 1.0000x reference)
import jax
import jax.numpy as jnp
from jax.experimental import pallas as pl
from jax.experimental.pallas import tpu as pltpu


def _fts_body(x_ref, w_ref, fts_ref):
    fts_ref[...] = jax.lax.dot_general(
        x_ref[...], w_ref[...],
        dimension_numbers=(((1,), (1,)), ((), ())),
        preferred_element_type=jnp.float32,
    )


def _agg_body(fts_ref, adj_ref, b_ref, out_ref):
    acc = jnp.dot(adj_ref[...], fts_ref[...], preferred_element_type=jnp.float32)
    out_ref[...] = jnp.maximum(acc + b_ref[...], 0.0)


def kernel(seq, adj, W, b):
    _, n, d_in = seq.shape
    d_out = W.shape[0]
    x = seq.reshape(n, d_in)
    bb = b.reshape(1, d_out)

    fts = pl.pallas_call(
        _fts_body,
        in_specs=[
            pl.BlockSpec((n, d_in), lambda: (0, 0)),
            pl.BlockSpec((d_out, d_in), lambda: (0, 0)),
        ],
        out_specs=pl.BlockSpec((n, d_out), lambda: (0, 0)),
        out_shape=jax.ShapeDtypeStruct((n, d_out), jnp.float32),
    )(x, W)

    bm = 400
    grid = (n // bm,)

    out = pl.pallas_call(
        _agg_body,
        grid=grid,
        in_specs=[
            pl.BlockSpec((n, d_out), lambda i: (0, 0)),
            pl.BlockSpec((bm, n), lambda i: (i, 0)),
            pl.BlockSpec((1, d_out), lambda i: (0, 0)),
        ],
        out_specs=pl.BlockSpec((bm, d_out), lambda i: (i, 0)),
        out_shape=jax.ShapeDtypeStruct((n, d_out), jnp.float32),
        compiler_params=pltpu.CompilerParams(
            dimension_semantics=("parallel",),
        ),
    )(fts, adj, bb)

    return out.reshape(1, n, d_out), fts.reshape(1, n, d_out)


# speedup vs baseline: 1.0358x; 1.0358x over previous
import jax
import jax.numpy as jnp
from jax.experimental import pallas as pl
from jax.experimental.pallas import tpu as pltpu


def _gcn_body(x_ref, adj_ref, w_ref, b_ref, out_ref, fts_ref, fts_acc):
    i = pl.program_id(0)

    @pl.when(i == 0)
    def _compute_fts():
        fts_acc[...] = jax.lax.dot_general(
            x_ref[...], w_ref[...],
            dimension_numbers=(((1,), (1,)), ((), ())),
            preferred_element_type=jnp.float32,
        )

    bm = out_ref.shape[0]
    fts_ref[...] = fts_acc[pl.ds(i * bm, bm), :]
    acc = jnp.dot(adj_ref[...], fts_acc[...], preferred_element_type=jnp.float32)
    out_ref[...] = jnp.maximum(acc + b_ref[...], 0.0)


def kernel(seq, adj, W, b):
    _, n, d_in = seq.shape
    d_out = W.shape[0]
    x = seq.reshape(n, d_in)
    bb = b.reshape(1, d_out)

    bm = 400
    grid = (n // bm,)

    out, fts = pl.pallas_call(
        _gcn_body,
        grid=grid,
        in_specs=[
            pl.BlockSpec((n, d_in), lambda i: (0, 0)),
            pl.BlockSpec((bm, n), lambda i: (i, 0)),
            pl.BlockSpec((d_out, d_in), lambda i: (0, 0)),
            pl.BlockSpec((1, d_out), lambda i: (0, 0)),
        ],
        out_specs=[
            pl.BlockSpec((bm, d_out), lambda i: (i, 0)),
            pl.BlockSpec((bm, d_out), lambda i: (i, 0)),
        ],
        out_shape=[
            jax.ShapeDtypeStruct((n, d_out), jnp.float32),
            jax.ShapeDtypeStruct((n, d_out), jnp.float32),
        ],
        scratch_shapes=[pltpu.VMEM((n, d_out), jnp.float32)],
    )(x, adj, W, bb)

    return out.reshape(1, n, d_out), fts.reshape(1, n, d_out)
